# baseline (speedup 1.0000x reference)
import functools

import jax
import jax.numpy as jnp
from jax import lax
from jax.experimental import pallas as pl
from jax.experimental.pallas import tpu as pltpu
from jax.experimental.pallas import tpu_sc as plsc


def _tc_body(img_ref, par_ref, color_ref, d_ref, *, ch, w):
    ii = pl.program_id(1)
    m = ch * w // 128
    wb = w // 128

    img = img_ref[:, 0].reshape(4, 3, m, 128)
    color_ref[...] = img.transpose(1, 2, 0, 3)

    pv = par_ref[0]

    def s(k):
        return pv[:, k].reshape(1, 4, 1)

    mi = jax.lax.broadcasted_iota(jnp.int32, (m, 1, 128), 0)
    li = jax.lax.broadcasted_iota(jnp.int32, (m, 1, 128), 2)
    col = ((mi % wb) * 128 + li).astype(jnp.float32)
    row = (ii * ch + mi // wb).astype(jnp.float32)
    xn = (col + 0.5 - s(2)) * s(0)
    yn = (row + 0.5 - s(3)) * s(1)
    dx = s(4) * xn + s(5) * yn + s(6)
    dy = s(7) * xn + s(8) * yn + s(9)
    dz = s(10) * xn + s(11) * yn + s(12)
    inv = jax.lax.rsqrt(dx * dx + dy * dy + dz * dz)
    d_ref[...] = jnp.stack([dx * inv, dy * inv, dz * inv], axis=0)


def _make_sc_ray_o(nh, rep, per_w, nh_per_v):
    mesh = plsc.VectorSubcoreMesh(core_axis_name="c", subcore_axis_name="s")

    @functools.partial(
        pl.kernel, mesh=mesh,
        out_type=jax.ShapeDtypeStruct((3, nh, 4, 128), jnp.float32),
        scratch_types=[
            pltpu.VMEM((rep, 4, 128), jnp.float32),
            pltpu.SemaphoreType.DMA,
        ],
    )
    def sc_ray_o(pat_hbm, o_hbm, buf, sem):
        wid = lax.axis_index("s") * 2 + lax.axis_index("c")
        nh0 = wid * per_w
        v = nh0 // nh_per_v
        for c in range(3):
            for j in range(rep):
                pltpu.sync_copy(pat_hbm.at[v, c], buf.at[j])
            for i in range(per_w // rep):
                pltpu.async_copy(
                    buf, o_hbm.at[c, pl.ds(nh0 + i * rep, rep)], sem
                ).wait()

    return sc_ray_o


def kernel(image, fxfycxcy, c2w, mv, mvp, depth, normal, index):
    b, v, c, h, w = image.shape
    ch = 128
    m = ch * w // 128
    nbk = h // ch
    n = v * h * w
    nh = n // 128

    f = fxfycxcy
    R = c2w[:, :, :3, :3].reshape(b, v, 9)
    t = c2w[:, :, :3, 3]
    params = jnp.concatenate(
        [1.0 / f[:, :, 0:1], 1.0 / f[:, :, 1:2], f[:, :, 2:4], R, t], axis=2
    ).transpose(1, 0, 2)

    out4 = jax.ShapeDtypeStruct((3, nh, b, 128), jnp.float32)
    grid = (v, nbk)

    def _shared_idx(vi, ii):
        return (0, vi * nbk + ii, 0, 0)

    color4, d4 = pl.pallas_call(
        functools.partial(_tc_body, ch=ch, w=w),
        grid=grid,
        in_specs=[
            pl.BlockSpec((4, 1, 3, ch, w), lambda vi, ii: (0, vi, 0, ii, 0)),
            pl.BlockSpec((1, 4, 16), lambda vi, ii: (vi, 0, 0)),
        ],
        out_specs=[
            pl.BlockSpec((3, m, 4, 128), _shared_idx),
            pl.BlockSpec((3, m, 4, 128), _shared_idx),
        ],
        out_shape=[out4, out4],
    )(image, params)

    pat = jnp.broadcast_to(t.transpose(1, 2, 0)[:, :, :, None], (v, 3, b, 128))
    sc_ray_o = _make_sc_ray_o(nh, 64, nh // 32, nh // v)
    o4 = sc_ray_o(pat)

    ray_color = color4.transpose(2, 1, 3, 0).reshape(b, n, 3)
    ray_o = o4.transpose(2, 1, 3, 0).reshape(b, n, 3)
    ray_d = d4.transpose(2, 1, 3, 0).reshape(b, n, 3)
    return (ray_color, ray_o, ray_d)


# speedup vs baseline: 1.8876x; 1.8876x over previous
import functools

import jax
import jax.numpy as jnp
from jax.experimental import pallas as pl


def _body(img_ref, par_ref, color_ref, o_ref, d_ref, *, ch, w):
    ii = pl.program_id(1)
    m = ch * w // 128
    wb = w // 128

    img = img_ref[:, 0].reshape(4, 3, m, 128)
    color_ref[...] = img.transpose(1, 2, 0, 3)

    pv = par_ref[0]

    def s(k):
        return pv[:, k].reshape(1, 4, 1)

    mi = jax.lax.broadcasted_iota(jnp.int32, (m, 1, 128), 0)
    li = jax.lax.broadcasted_iota(jnp.int32, (m, 1, 128), 2)
    col = ((mi % wb) * 128 + li).astype(jnp.float32)
    row = (ii * ch + mi // wb).astype(jnp.float32)
    xn = (col + 0.5 - s(2)) * s(0)
    yn = (row + 0.5 - s(3)) * s(1)
    dx = s(4) * xn + s(5) * yn + s(6)
    dy = s(7) * xn + s(8) * yn + s(9)
    dz = s(10) * xn + s(11) * yn + s(12)
    inv = jax.lax.rsqrt(dx * dx + dy * dy + dz * dz)
    d_ref[...] = jnp.stack([dx * inv, dy * inv, dz * inv], axis=0)
    shape = (m, 4, 128)
    o_ref[...] = jnp.stack(
        [jnp.broadcast_to(s(13), shape), jnp.broadcast_to(s(14), shape),
         jnp.broadcast_to(s(15), shape)], axis=0)


def kernel(image, fxfycxcy, c2w, mv, mvp, depth, normal, index):
    b, v, c, h, w = image.shape
    ch = 128
    m = ch * w // 128
    nbk = h // ch
    n = v * h * w

    f = fxfycxcy
    R = c2w[:, :, :3, :3].reshape(b, v, 9)
    t = c2w[:, :, :3, 3]
    params = jnp.concatenate(
        [1.0 / f[:, :, 0:1], 1.0 / f[:, :, 1:2], f[:, :, 2:4], R, t], axis=2
    ).transpose(1, 0, 2)

    out4 = jax.ShapeDtypeStruct((3, n // 128, b, 128), jnp.float32)
    grid = (v, nbk)

    def _shared_idx(vi, ii):
        return (0, vi * nbk + ii, 0, 0)

    color4, o4, d4 = pl.pallas_call(
        functools.partial(_body, ch=ch, w=w),
        grid=grid,
        in_specs=[
            pl.BlockSpec((4, 1, 3, ch, w), lambda vi, ii: (0, vi, 0, ii, 0)),
            pl.BlockSpec((1, 4, 16), lambda vi, ii: (vi, 0, 0)),
        ],
        out_specs=[
            pl.BlockSpec((3, m, 4, 128), _shared_idx),
            pl.BlockSpec((3, m, 4, 128), _shared_idx),
            pl.BlockSpec((3, m, 4, 128), _shared_idx),
        ],
        out_shape=[out4, out4, out4],
    )(image, params)

    ray_color = color4.transpose(2, 1, 3, 0).reshape(b, n, 3)
    ray_o = o4.transpose(2, 1, 3, 0).reshape(b, n, 3)
    ray_d = d4.transpose(2, 1, 3, 0).reshape(b, n, 3)
    return (ray_color, ray_o, ray_d)
